# Initial kernel scaffold; baseline (speedup 1.0000x reference)
import jax
import jax.numpy as jnp
from jax.experimental import pallas as pl


def kernel(x, edge_index, W0, att_src0, att_dst0, b0, W1, att_src1, att_dst1, b1, clsW, clsb):
    raise NotImplementedError("write your pallas kernel here")


# speedup vs baseline: 22.8077x; 22.8077x over previous
import functools

import jax
import jax.numpy as jnp
from jax import lax
from jax.experimental import pallas as pl
from jax.experimental.pallas import tpu as pltpu
from jax.experimental.pallas import tpu_sc as plsc

N = 10000
E = 320000
D = 128
NHID = 128
NCLS = 40

NC = 2
NS = 16
NW = NC * NS
K = 64

N_PAD = 10240
AA_PAD = 10016
ROWS_PER_TILE = N_PAD // NS
E_FULL = E + N
PER_TILE = ((E_FULL + NW * K - 1) // (NW * K)) * K
E_PAD = PER_TILE * NW
N_CHUNKS = PER_TILE // K

_BLK = 1000


def _mm_att_body(x_ref, w_ref, asr_ref, adr_ref, h_ref, aa_ref):
    h = jnp.dot(x_ref[...], w_ref[...], preferred_element_type=jnp.float32)
    h_ref[...] = h
    aa_ref[:, 0:1] = jnp.sum(h * asr_ref[...], axis=1, keepdims=True)
    aa_ref[:, 1:2] = jnp.sum(h * adr_ref[...], axis=1, keepdims=True)


def _mm_att(x, W, a_src, a_dst):
    n = x.shape[0]
    grid = n // _BLK
    return pl.pallas_call(
        _mm_att_body,
        grid=(grid,),
        in_specs=[
            pl.BlockSpec((_BLK, D), lambda i: (i, 0)),
            pl.BlockSpec((D, NHID), lambda i: (0, 0)),
            pl.BlockSpec((1, NHID), lambda i: (0, 0)),
            pl.BlockSpec((1, NHID), lambda i: (0, 0)),
        ],
        out_specs=[
            pl.BlockSpec((_BLK, NHID), lambda i: (i, 0)),
            pl.BlockSpec((_BLK, 2), lambda i: (i, 0)),
        ],
        out_shape=[
            jax.ShapeDtypeStruct((n, NHID), jnp.float32),
            jax.ShapeDtypeStruct((n, 2), jnp.float32),
        ],
    )(x, W, a_src, a_dst)


def _norm_mm_att_body(na_ref, nb_ref, da_ref, db_ref, b_ref, w_ref, asr_ref,
                      adr_ref, h_ref, aa_ref):
    den = da_ref[:, 0:1] + db_ref[:, 0:1]
    hin = (na_ref[...] + nb_ref[...]) / (den + 1e-16) + b_ref[...]
    hin = jnp.maximum(hin, 0.0)
    h = jnp.dot(hin, w_ref[...], preferred_element_type=jnp.float32)
    h_ref[...] = h
    aa_ref[:, 0:1] = jnp.sum(h * asr_ref[...], axis=1, keepdims=True)
    aa_ref[:, 1:2] = jnp.sum(h * adr_ref[...], axis=1, keepdims=True)


def _norm_mm_att(num_a, num_b, den_a, den_b, b, W, a_src, a_dst):
    grid = N // _BLK
    return pl.pallas_call(
        _norm_mm_att_body,
        grid=(grid,),
        in_specs=[
            pl.BlockSpec((_BLK, NHID), lambda i: (i, 0)),
            pl.BlockSpec((_BLK, NHID), lambda i: (i, 0)),
            pl.BlockSpec((_BLK, 16), lambda i: (i, 0)),
            pl.BlockSpec((_BLK, 16), lambda i: (i, 0)),
            pl.BlockSpec((1, NHID), lambda i: (0, 0)),
            pl.BlockSpec((NHID, NHID), lambda i: (0, 0)),
            pl.BlockSpec((1, NHID), lambda i: (0, 0)),
            pl.BlockSpec((1, NHID), lambda i: (0, 0)),
        ],
        out_specs=[
            pl.BlockSpec((_BLK, NHID), lambda i: (i, 0)),
            pl.BlockSpec((_BLK, 2), lambda i: (i, 0)),
        ],
        out_shape=[
            jax.ShapeDtypeStruct((N, NHID), jnp.float32),
            jax.ShapeDtypeStruct((N, 2), jnp.float32),
        ],
    )(num_a, num_b, den_a, den_b, b, W, a_src, a_dst)


def _norm_cls_body(na_ref, nb_ref, da_ref, db_ref, b_ref, w_ref, cb_ref,
                   out_ref):
    den = da_ref[:, 0:1] + db_ref[:, 0:1]
    hin = (na_ref[...] + nb_ref[...]) / (den + 1e-16) + b_ref[...]
    hin = jnp.maximum(hin, 0.0)
    out_ref[...] = (
        jnp.dot(hin, w_ref[...], preferred_element_type=jnp.float32)
        + cb_ref[...]
    )


def _norm_cls(num_a, num_b, den_a, den_b, b, clsW, clsb):
    grid = N // _BLK
    return pl.pallas_call(
        _norm_cls_body,
        grid=(grid,),
        in_specs=[
            pl.BlockSpec((_BLK, NHID), lambda i: (i, 0)),
            pl.BlockSpec((_BLK, NHID), lambda i: (i, 0)),
            pl.BlockSpec((_BLK, 16), lambda i: (i, 0)),
            pl.BlockSpec((_BLK, 16), lambda i: (i, 0)),
            pl.BlockSpec((1, NHID), lambda i: (0, 0)),
            pl.BlockSpec((NHID, NCLS), lambda i: (0, 0)),
            pl.BlockSpec((1, NCLS), lambda i: (0, 0)),
        ],
        out_specs=pl.BlockSpec((_BLK, NCLS), lambda i: (i, 0)),
        out_shape=jax.ShapeDtypeStruct((N, NCLS), jnp.float32),
    )(num_a, num_b, den_a, den_b, b, clsW, clsb)


def _make_sc_edge(interpret=False):
    mesh = plsc.VectorSubcoreMesh(
        core_axis_name="c", subcore_axis_name="s",
        num_cores=NC, num_subcores=NS)
    return pl.kernel(
        _sc_edge_body,
        out_type=[
            jax.ShapeDtypeStruct((NC, N_PAD, NHID), jnp.float32),
            jax.ShapeDtypeStruct((NC, N_PAD, 16), jnp.float32),
        ],
        mesh=mesh,
        compiler_params=pltpu.CompilerParams(
            needs_layout_passes=False, use_tc_tiling_on_sc=False),
        interpret=interpret,
        scratch_types=[
            pltpu.VMEM_SHARED((N_PAD, NHID), jnp.float32),
            pltpu.VMEM_SHARED((N_PAD, 16), jnp.float32),
            pltpu.VMEM((AA_PAD * 2,), jnp.float32),
            [pltpu.VMEM((K,), jnp.int32) for _ in range(2)],
            [pltpu.VMEM((K,), jnp.int32) for _ in range(2)],
            [pltpu.VMEM((K, NHID), jnp.float32) for _ in range(2)],
            [pltpu.VMEM((K, 16), jnp.float32) for _ in range(2)],
            [pltpu.SemaphoreType.DMA for _ in range(2)],
            [pltpu.SemaphoreType.DMA for _ in range(2)],
            [pltpu.SemaphoreType.DMA for _ in range(2)],
        ],
    )


def _sc_edge_body(aa_hbm, h_hbm, src_hbm, dst_hbm, znum_hbm, zden_hbm,
                  num_out, den_out,
                  num_sp, den_sp, aa_v, sidx, didx, rows, denp,
                  gsem, ssem, dsem):
    c = lax.axis_index("c")
    s = lax.axis_index("s")
    wid = c * NS + s

    zf16 = jnp.zeros((16,), jnp.float32)
    zi16 = jnp.zeros((16,), jnp.int32)
    iota16 = lax.iota(jnp.int32, 16)

    for b in range(2):
        for r in range(K):
            denp[b][r, :] = zf16

    row0 = s * ROWS_PER_TILE
    pltpu.sync_copy(znum_hbm, num_sp.at[pl.ds(row0, ROWS_PER_TILE)])
    pltpu.sync_copy(zden_hbm, den_sp.at[pl.ds(row0, ROWS_PER_TILE)])

    pltpu.sync_copy(aa_hbm, aa_v)

    plsc.subcore_barrier()

    def _chunk(t, b, drain):
        if drain:
            pltpu.make_async_copy(rows[b], num_sp.at[didx[b]], ssem[b]).wait()
            pltpu.make_async_copy(denp[b], den_sp.at[didx[b]], dsem[b]).wait()
        base = wid * PER_TILE + t * K
        pltpu.sync_copy(src_hbm.at[pl.ds(base, K)], sidx[b])
        pltpu.sync_copy(dst_hbm.at[pl.ds(base, K)], didx[b])
        gather = pltpu.async_copy(h_hbm.at[sidx[b]], rows[b], gsem[b])

        ws = []
        for g in range(K // 16):
            s16 = sidx[b][pl.ds(g * 16, 16)]
            d16 = didx[b][pl.ds(g * 16, 16)]
            a = (plsc.load_gather(aa_v, [s16 * 2])
                 + plsc.load_gather(aa_v, [d16 * 2 + 1]))
            a = jnp.maximum(a, 0.2 * a)
            w16 = jnp.exp(a)
            ws.append(w16)
            plsc.store_scatter(denp[b], [iota16 + g * 16, zi16], w16)

        gather.wait()

        for k in range(K):
            g, j = divmod(k, 16)
            wspl = jnp.full((16,), ws[g][j], jnp.float32)
            for cg in range(NHID // 16):
                rows[b][k, pl.ds(cg * 16, 16)] = (
                    rows[b][k, pl.ds(cg * 16, 16)] * wspl)

        pltpu.async_copy(rows[b], num_sp.at[didx[b]], ssem[b], add=True)
        pltpu.async_copy(denp[b], den_sp.at[didx[b]], dsem[b], add=True)

    for b in range(2):
        _chunk(jnp.int32(b), b, drain=False)

    def _pair(g, carry):
        for b in range(2):
            _chunk(g * 2 + b, b, drain=True)
        return carry

    lax.fori_loop(1, N_CHUNKS // 2, _pair, 0)

    for b in range(2):
        pltpu.make_async_copy(rows[b], num_sp.at[didx[b]], ssem[b]).wait()
        pltpu.make_async_copy(denp[b], den_sp.at[didx[b]], dsem[b]).wait()

    plsc.subcore_barrier()

    pltpu.sync_copy(num_sp.at[pl.ds(row0, ROWS_PER_TILE)],
                    num_out.at[c, pl.ds(row0, ROWS_PER_TILE)])
    pltpu.sync_copy(den_sp.at[pl.ds(row0, ROWS_PER_TILE)],
                    den_out.at[c, pl.ds(row0, ROWS_PER_TILE)])


@functools.lru_cache(maxsize=None)
def _get_sc_edge():
    return _make_sc_edge()


def kernel(x, edge_index, W0, att_src0, att_dst0, b0, W1, att_src1, att_dst1,
           b1, clsW, clsb):
    src = edge_index[0].astype(jnp.int32)
    dst = edge_index[1].astype(jnp.int32)
    loops = jnp.arange(N, dtype=jnp.int32)
    pad = E_PAD - E_FULL
    src_full = jnp.concatenate([src, loops, jnp.zeros((pad,), jnp.int32)])
    dst_full = jnp.concatenate([
        jnp.where(src != dst, dst, N), loops,
        jnp.full((pad,), N, jnp.int32)])

    a_src0 = att_src0.reshape(1, NHID)
    a_dst0 = att_dst0.reshape(1, NHID)
    a_src1 = att_src1.reshape(1, NHID)
    a_dst1 = att_dst1.reshape(1, NHID)

    znum = jnp.zeros((ROWS_PER_TILE, NHID), jnp.float32)
    zden = jnp.zeros((ROWS_PER_TILE, 16), jnp.float32)

    sc_edge = _get_sc_edge()
    h0, aa0 = _mm_att(x, W0, a_src0, a_dst0)
    aa0p = jnp.pad(aa0, ((0, AA_PAD - N), (0, 0))).reshape(-1)
    num0, den0 = sc_edge(aa0p, h0, src_full, dst_full, znum, zden)

    h1, aa1 = _norm_mm_att(num0[0, :N], num0[1, :N], den0[0, :N], den0[1, :N],
                           b0.reshape(1, NHID), W1, a_src1, a_dst1)
    aa1p = jnp.pad(aa1, ((0, AA_PAD - N), (0, 0))).reshape(-1)
    num1, den1 = sc_edge(aa1p, h1, src_full, dst_full, znum, zden)

    return _norm_cls(num1[0, :N], num1[1, :N], den1[0, :N], den1[1, :N],
                     b1.reshape(1, NHID), clsW, clsb.reshape(1, NCLS))


# speedup vs baseline: 29.2503x; 1.2825x over previous
import functools

import jax
import jax.numpy as jnp
from jax import lax
from jax.experimental import pallas as pl
from jax.experimental.pallas import tpu as pltpu
from jax.experimental.pallas import tpu_sc as plsc

N = 10000
E = 320000
D = 128
NHID = 128
NCLS = 40

NC = 2
NS = 16
NW = NC * NS
K = 48
NB = 3

N_PAD = 10240
AA_PAD = 10016
ROWS_PER_TILE = N_PAD // NS
E_FULL = E + N
SUP = 6
N_CHUNKS = 216
NSUP = N_CHUNKS // SUP
PER_TILE = N_CHUNKS * K
E_PAD = PER_TILE * NW

_BLK = 1000


def _mm_att_body(x_ref, w_ref, asr_ref, adr_ref, h_ref, aa_ref):
    h = jnp.dot(x_ref[...], w_ref[...], preferred_element_type=jnp.float32)
    h_ref[...] = h
    aa_ref[:, 0:1] = jnp.sum(h * asr_ref[...], axis=1, keepdims=True)
    aa_ref[:, 1:2] = jnp.sum(h * adr_ref[...], axis=1, keepdims=True)


def _mm_att(x, W, a_src, a_dst):
    n = x.shape[0]
    grid = n // _BLK
    return pl.pallas_call(
        _mm_att_body,
        grid=(grid,),
        in_specs=[
            pl.BlockSpec((_BLK, D), lambda i: (i, 0)),
            pl.BlockSpec((D, NHID), lambda i: (0, 0)),
            pl.BlockSpec((1, NHID), lambda i: (0, 0)),
            pl.BlockSpec((1, NHID), lambda i: (0, 0)),
        ],
        out_specs=[
            pl.BlockSpec((_BLK, NHID), lambda i: (i, 0)),
            pl.BlockSpec((_BLK, 2), lambda i: (i, 0)),
        ],
        out_shape=[
            jax.ShapeDtypeStruct((n, NHID), jnp.float32),
            jax.ShapeDtypeStruct((n, 2), jnp.float32),
        ],
    )(x, W, a_src, a_dst)


def _norm_mm_att_body(na_ref, nb_ref, da_ref, db_ref, b_ref, w_ref, asr_ref,
                      adr_ref, h_ref, aa_ref):
    den = da_ref[:, 0:1] + db_ref[:, 0:1]
    hin = (na_ref[...] + nb_ref[...]) / (den + 1e-16) + b_ref[...]
    hin = jnp.maximum(hin, 0.0)
    h = jnp.dot(hin, w_ref[...], preferred_element_type=jnp.float32)
    h_ref[...] = h
    aa_ref[:, 0:1] = jnp.sum(h * asr_ref[...], axis=1, keepdims=True)
    aa_ref[:, 1:2] = jnp.sum(h * adr_ref[...], axis=1, keepdims=True)


def _norm_mm_att(num_a, num_b, den_a, den_b, b, W, a_src, a_dst):
    grid = N // _BLK
    return pl.pallas_call(
        _norm_mm_att_body,
        grid=(grid,),
        in_specs=[
            pl.BlockSpec((_BLK, NHID), lambda i: (i, 0)),
            pl.BlockSpec((_BLK, NHID), lambda i: (i, 0)),
            pl.BlockSpec((_BLK, 8), lambda i: (i, 0)),
            pl.BlockSpec((_BLK, 8), lambda i: (i, 0)),
            pl.BlockSpec((1, NHID), lambda i: (0, 0)),
            pl.BlockSpec((NHID, NHID), lambda i: (0, 0)),
            pl.BlockSpec((1, NHID), lambda i: (0, 0)),
            pl.BlockSpec((1, NHID), lambda i: (0, 0)),
        ],
        out_specs=[
            pl.BlockSpec((_BLK, NHID), lambda i: (i, 0)),
            pl.BlockSpec((_BLK, 2), lambda i: (i, 0)),
        ],
        out_shape=[
            jax.ShapeDtypeStruct((N, NHID), jnp.float32),
            jax.ShapeDtypeStruct((N, 2), jnp.float32),
        ],
    )(num_a, num_b, den_a, den_b, b, W, a_src, a_dst)


def _norm_cls_body(na_ref, nb_ref, da_ref, db_ref, b_ref, w_ref, cb_ref,
                   out_ref):
    den = da_ref[:, 0:1] + db_ref[:, 0:1]
    hin = (na_ref[...] + nb_ref[...]) / (den + 1e-16) + b_ref[...]
    hin = jnp.maximum(hin, 0.0)
    out_ref[...] = (
        jnp.dot(hin, w_ref[...], preferred_element_type=jnp.float32)
        + cb_ref[...]
    )


def _norm_cls(num_a, num_b, den_a, den_b, b, clsW, clsb):
    grid = N // _BLK
    return pl.pallas_call(
        _norm_cls_body,
        grid=(grid,),
        in_specs=[
            pl.BlockSpec((_BLK, NHID), lambda i: (i, 0)),
            pl.BlockSpec((_BLK, NHID), lambda i: (i, 0)),
            pl.BlockSpec((_BLK, 8), lambda i: (i, 0)),
            pl.BlockSpec((_BLK, 8), lambda i: (i, 0)),
            pl.BlockSpec((1, NHID), lambda i: (0, 0)),
            pl.BlockSpec((NHID, NCLS), lambda i: (0, 0)),
            pl.BlockSpec((1, NCLS), lambda i: (0, 0)),
        ],
        out_specs=pl.BlockSpec((_BLK, NCLS), lambda i: (i, 0)),
        out_shape=jax.ShapeDtypeStruct((N, NCLS), jnp.float32),
    )(num_a, num_b, den_a, den_b, b, clsW, clsb)


def _make_sc_edge(interpret=False):
    mesh = plsc.VectorSubcoreMesh(
        core_axis_name="c", subcore_axis_name="s",
        num_cores=NC, num_subcores=NS)
    return pl.kernel(
        _sc_edge_body,
        out_type=[
            jax.ShapeDtypeStruct((NC, N_PAD, NHID), jnp.float32),
            jax.ShapeDtypeStruct((NC, N_PAD, 8), jnp.float32),
        ],
        mesh=mesh,
        compiler_params=pltpu.CompilerParams(
            needs_layout_passes=False, use_tc_tiling_on_sc=False),
        interpret=interpret,
        scratch_types=[
            pltpu.VMEM_SHARED((N_PAD, NHID), jnp.float32),
            pltpu.VMEM_SHARED((N_PAD, 8), jnp.float32),
            pltpu.VMEM((AA_PAD * 2,), jnp.float32),
            pltpu.VMEM((SUP, K), jnp.int32),
            pltpu.VMEM((SUP, K), jnp.int32),
            [pltpu.VMEM((K, NHID), jnp.float32) for _ in range(NB)],
            [pltpu.VMEM((K, 8), jnp.float32) for _ in range(NB)],
            [pltpu.VMEM((K,), jnp.float32) for _ in range(NB)],
            [pltpu.SemaphoreType.DMA for _ in range(NB)],
            [pltpu.SemaphoreType.DMA for _ in range(NB)],
            [pltpu.SemaphoreType.DMA for _ in range(NB)],
        ],
    )


def _sc_edge_body(aa_hbm, h_hbm, src_hbm, dst_hbm, znum_hbm, zden_hbm,
                  num_out, den_out,
                  num_sp, den_sp, aa_v, sidxB, didxB, rows, denp, w_v,
                  gsem, ssem, dsem):
    c = lax.axis_index("c")
    s = lax.axis_index("s")
    wid = c * NS + s

    zi16 = jnp.zeros((16,), jnp.int32)
    iota16 = lax.iota(jnp.int32, 16)

    row0 = s * ROWS_PER_TILE
    pltpu.sync_copy(znum_hbm, num_sp.at[pl.ds(row0, ROWS_PER_TILE)])
    pltpu.sync_copy(zden_hbm, den_sp.at[pl.ds(row0, ROWS_PER_TILE)])

    pltpu.sync_copy(aa_hbm, aa_v)

    plsc.subcore_barrier()


    def load_super(si):
        r0 = wid * N_CHUNKS + si * SUP
        pltpu.sync_copy(src_hbm.at[pl.ds(r0, SUP)], sidxB)
        pltpu.sync_copy(dst_hbm.at[pl.ds(r0, SUP)], didxB)

    def drain(rb, j):
        pltpu.make_async_copy(rows[rb], num_sp.at[didxB.at[j]],
                              ssem[rb]).wait()
        pltpu.make_async_copy(denp[rb], den_sp.at[didxB.at[j]],
                              dsem[rb]).wait()

    def gstage(rb, j):
        if j >= NB:
            drain(rb, j - NB)
        pltpu.async_copy(h_hbm.at[sidxB.at[j]], rows[rb], gsem[rb])
        for g in range(K // 16):
            s16 = sidxB[j, pl.ds(g * 16, 16)]
            d16 = didxB[j, pl.ds(g * 16, 16)]
            a = (plsc.load_gather(aa_v, [s16 * 2])
                 + plsc.load_gather(aa_v, [d16 * 2 + 1]))
            a = jnp.maximum(a, 0.2 * a)
            w16 = jnp.exp(a)
            w_v[rb][pl.ds(g * 16, 16)] = w16
            plsc.store_scatter(denp[rb], [iota16 + g * 16, zi16], w16)

    def cstage(rb, j):
        pltpu.make_async_copy(h_hbm.at[sidxB.at[j]], rows[rb],
                              gsem[rb]).wait()
        for g in range(K // 16):
            wg = w_v[rb][pl.ds(g * 16, 16)]
            for jj in range(16):
                k = g * 16 + jj
                wspl = jnp.full((16,), wg[jj], jnp.float32)
                for cg in range(NHID // 16):
                    rows[rb][k, pl.ds(cg * 16, 16)] = (
                        rows[rb][k, pl.ds(cg * 16, 16)] * wspl)
        pltpu.async_copy(rows[rb], num_sp.at[didxB.at[j]], ssem[rb], add=True)
        pltpu.async_copy(denp[rb], den_sp.at[didxB.at[j]], dsem[rb], add=True)

    load_super(jnp.int32(0))
    gstage(0, 0)
    for j in range(1, SUP):
        gstage(j % NB, j)
        cstage((j - 1) % NB, j - 1)

    def _super(si, carry):
        cstage((SUP - 1) % NB, SUP - 1)
        for j in range(SUP - NB, SUP):
            drain(j % NB, j)
        load_super(si)
        gstage(0, 0)
        for j in range(1, SUP):
            gstage(j % NB, j)
            cstage((j - 1) % NB, j - 1)
        return carry

    lax.fori_loop(1, NSUP, _super, 0)

    cstage((SUP - 1) % NB, SUP - 1)
    for j in range(SUP - NB, SUP):
        drain(j % NB, j)

    plsc.subcore_barrier()

    pltpu.sync_copy(num_sp.at[pl.ds(row0, ROWS_PER_TILE)],
                    num_out.at[c, pl.ds(row0, ROWS_PER_TILE)])
    pltpu.sync_copy(den_sp.at[pl.ds(row0, ROWS_PER_TILE)],
                    den_out.at[c, pl.ds(row0, ROWS_PER_TILE)])


@functools.lru_cache(maxsize=None)
def _get_sc_edge():
    return _make_sc_edge()


def kernel(x, edge_index, W0, att_src0, att_dst0, b0, W1, att_src1, att_dst1,
           b1, clsW, clsb):
    src = edge_index[0].astype(jnp.int32)
    dst = edge_index[1].astype(jnp.int32)
    loops = jnp.arange(N, dtype=jnp.int32)
    pad = E_PAD - E_FULL
    src_full = jnp.concatenate(
        [src, loops, jnp.zeros((pad,), jnp.int32)]).reshape(-1, K)
    dst_full = jnp.concatenate([
        jnp.where(src != dst, dst, N), loops,
        jnp.full((pad,), N, jnp.int32)]).reshape(-1, K)

    a_src0 = att_src0.reshape(1, NHID)
    a_dst0 = att_dst0.reshape(1, NHID)
    a_src1 = att_src1.reshape(1, NHID)
    a_dst1 = att_dst1.reshape(1, NHID)

    znum = jnp.zeros((ROWS_PER_TILE, NHID), jnp.float32)
    zden = jnp.zeros((ROWS_PER_TILE, 8), jnp.float32)

    sc_edge = _get_sc_edge()
    h0, aa0 = _mm_att(x, W0, a_src0, a_dst0)
    aa0p = jnp.pad(aa0, ((0, AA_PAD - N), (0, 0))).reshape(-1)
    num0, den0 = sc_edge(aa0p, h0, src_full, dst_full, znum, zden)

    h1, aa1 = _norm_mm_att(num0[0, :N], num0[1, :N], den0[0, :N], den0[1, :N],
                           b0.reshape(1, NHID), W1, a_src1, a_dst1)
    aa1p = jnp.pad(aa1, ((0, AA_PAD - N), (0, 0))).reshape(-1)
    num1, den1 = sc_edge(aa1p, h1, src_full, dst_full, znum, zden)

    return _norm_cls(num1[0, :N], num1[1, :N], den1[0, :N], den1[1, :N],
                     b1.reshape(1, NHID), clsW, clsb.reshape(1, NCLS))
